# baseline (speedup 1.0000x reference)
import functools

import jax
import jax.numpy as jnp
import numpy as np
from jax import lax
from jax.experimental import pallas as pl
from jax.experimental.pallas import tpu as pltpu
from jax.experimental.pallas import tpu_sc as plsc

NUM_NODES = 116
KEEP_RATIO = 0.6
N_NODES = 3712
NPAD = 3840
DUMMY = 3712


def _make_pair_agg(P, D, C):
    assert P % (32 * C) == 0
    NT = P // (32 * C)
    mesh = plsc.VectorSubcoreMesh(core_axis_name="c", subcore_axis_name="s")
    rows_per = NPAD // 16

    @functools.partial(
        pl.kernel, mesh=mesh,
        compiler_params=pltpu.CompilerParams(needs_layout_passes=False),
        out_type=[
            jax.ShapeDtypeStruct((2, NPAD, D), jnp.float32),
            jax.ShapeDtypeStruct((2, NPAD, 16), jnp.float32),
        ],
        scratch_types=[
            pltpu.VMEM((C,), jnp.int32),
            pltpu.VMEM((C,), jnp.int32),
            pltpu.VMEM((C, D), jnp.float32),
            pltpu.VMEM((C, D), jnp.float32),
            pltpu.VMEM((C, 16), jnp.float32),
            pltpu.VMEM_SHARED((NPAD, D), jnp.float32),
            pltpu.VMEM_SHARED((NPAD, 16), jnp.float32),
            pltpu.SemaphoreType.DMA,
            pltpu.SemaphoreType.DMA,
        ],
    )
    def agg(src_hbm, dst_hbm, x_hbm, zfeat_hbm, zcnt_hbm, ones_hbm,
            acc_out, cnt_out,
            idx_s, idx_d, rows_a, rows_b, ones_v, acc_sh, cnt_sh, sem1, sem2):
        c = lax.axis_index("c")
        s = lax.axis_index("s")
        wid = c * 16 + s
        pltpu.sync_copy(zfeat_hbm.at[pl.ds(s * rows_per, rows_per)],
                        acc_sh.at[pl.ds(s * rows_per, rows_per)])
        pltpu.sync_copy(zcnt_hbm.at[pl.ds(s * rows_per, rows_per)],
                        cnt_sh.at[pl.ds(s * rows_per, rows_per)])
        pltpu.sync_copy(ones_hbm, ones_v)
        plsc.subcore_barrier()
        base0 = wid * (NT * C)

        def chunk(i, carry):
            base = base0 + i * C
            pltpu.sync_copy(src_hbm.at[pl.ds(base, C)], idx_s)
            pltpu.sync_copy(dst_hbm.at[pl.ds(base, C)], idx_d)
            cp1 = pltpu.async_copy(x_hbm.at[idx_s], rows_a, sem1)
            cp2 = pltpu.async_copy(x_hbm.at[idx_d], rows_b, sem2)
            cp1.wait()
            cp2.wait()
            pltpu.sync_copy(rows_a, acc_sh.at[idx_d], add=True)
            pltpu.sync_copy(rows_b, acc_sh.at[idx_s], add=True)
            pltpu.sync_copy(ones_v, cnt_sh.at[idx_d], add=True)
            pltpu.sync_copy(ones_v, cnt_sh.at[idx_s], add=True)
            return carry

        lax.fori_loop(0, NT, chunk, 0)
        plsc.subcore_barrier()
        pltpu.sync_copy(acc_sh.at[pl.ds(s * rows_per, rows_per)],
                        acc_out.at[c].at[pl.ds(s * rows_per, rows_per)])
        pltpu.sync_copy(cnt_sh.at[pl.ds(s * rows_per, rows_per)],
                        cnt_out.at[c].at[pl.ds(s * rows_per, rows_per)])

    return agg


def _pair_agg(src, dst, x_pad, D, C=160):
    P = src.shape[0]
    zfeat = jnp.zeros((NPAD, D), jnp.float32)
    zcnt = jnp.zeros((NPAD, 16), jnp.float32)
    ones = jnp.zeros((C, 16), jnp.float32).at[:, 0].set(1.0)
    return _make_pair_agg(P, D, C)(src, dst, x_pad, zfeat, zcnt, ones)


ET = 102400
GMUL = 565
TOPK_CC = 2048
KCAP = 81920
SENT = -2147483648


def _count_ge(pbuf, nv, t):
    def body(i, acc):
        v = pbuf[pl.ds(i * 16, 16)]
        return acc + jnp.where(v >= t, 1, 0)

    accv = lax.fori_loop(0, nv, body, jnp.zeros((16,), jnp.int32))
    return jnp.sum(accv)


def _make_topk_stats():
    mesh = plsc.VectorSubcoreMesh(core_axis_name="c", subcore_axis_name="s")
    NCH = ET // TOPK_CC

    @functools.partial(
        pl.kernel, mesh=mesh,
        compiler_params=pltpu.CompilerParams(needs_layout_passes=False),
        out_type=[jax.ShapeDtypeStruct((32, 16), jnp.int32)],
        scratch_types=[
            pltpu.VMEM((ET + 32,), jnp.int32),
            pltpu.VMEM((TOPK_CC,), jnp.int32),
            pltpu.VMEM((TOPK_CC,), jnp.int32),
            pltpu.VMEM((16,), jnp.int32),
        ],
    )
    def stats(src_hbm, pbits_hbm, info_out, pbuf, srcv, probv, infov):
        g = lax.axis_index("c") * 16 + lax.axis_index("s")
        lanes = lax.iota(jnp.int32, 16)
        dump = jnp.int32(ET + 31)

        def chunk(i, cnt):
            base = i * TOPK_CC
            pltpu.sync_copy(src_hbm.at[pl.ds(base, TOPK_CC)], srcv)
            pltpu.sync_copy(pbits_hbm.at[pl.ds(base, TOPK_CC)], probv)

            def vreg(j, cnt):
                s = srcv[pl.ds(j * 16, 16)]
                pb = probv[pl.ds(j * 16, 16)]
                m = lax.shift_right_logical(s * GMUL, 16) == g
                pc = plsc.cumsum(jnp.where(m, 1, 0))
                plsc.store_scatter(pbuf, [jnp.where(m, cnt + pc - 1, dump)], pb)
                return cnt + jnp.max(pc)

            return lax.fori_loop(0, TOPK_CC // 16, vreg, cnt, unroll=4)

        cnt = lax.fori_loop(0, NCH, chunk, jnp.int32(0))
        pbuf[pl.ds(cnt, 16)] = jnp.full((16,), SENT, jnp.int32)
        nv = (cnt + 15) // 16
        k = jnp.maximum(1, (jnp.float32(KEEP_RATIO) * cnt.astype(jnp.float32)).astype(jnp.int32))

        def bstep(_, lohi):
            lo, hi = lohi
            mid = (lo + hi) // 2
            go = hi - lo > 1
            c = _count_ge(pbuf, nv, mid)
            take = jnp.logical_and(go, c >= k)
            lo2 = jnp.where(take, mid, lo)
            hi2 = jnp.where(jnp.logical_and(go, jnp.logical_not(c >= k)), mid, hi)
            return lo2, hi2

        vk, _ = lax.fori_loop(0, 31, bstep, (jnp.int32(0), jnp.int32(0x3F800001)))
        c_gt = _count_ge(pbuf, nv, vk + 1)
        kc = jnp.where(cnt > 0, k, 0)
        ntie = k - c_gt
        row = jnp.where(lanes == 0, kc,
                        jnp.where(lanes == 1, vk,
                                  jnp.where(lanes == 2, ntie, 0)))
        infov[...] = row
        pltpu.sync_copy(infov, info_out.at[g])

    return stats


def _make_topk_emit():
    mesh = plsc.VectorSubcoreMesh(core_axis_name="c", subcore_axis_name="s")
    NCH = ET // TOPK_CC
    SCAP = 1040

    @functools.partial(
        pl.kernel, mesh=mesh,
        compiler_params=pltpu.CompilerParams(needs_layout_passes=False),
        out_type=[
            jax.ShapeDtypeStruct((KCAP,), jnp.int32),
            jax.ShapeDtypeStruct((KCAP,), jnp.int32),
        ],
        scratch_types=[
            pltpu.VMEM((TOPK_CC,), jnp.int32),
            pltpu.VMEM((TOPK_CC,), jnp.int32),
            pltpu.VMEM((TOPK_CC,), jnp.int32),
            pltpu.VMEM((SCAP,), jnp.int32),
            pltpu.VMEM((SCAP,), jnp.int32),
            pltpu.VMEM((32, 16), jnp.int32),
        ],
    )
    def emit(src_hbm, dst_hbm, pbits_hbm, info_hbm, ksrc_out, kdst_out,
             srcv, dstv, probv, stag_s, stag_d, infov):
        g = lax.axis_index("c") * 16 + lax.axis_index("s")
        lanes = lax.iota(jnp.int32, 16)
        zeros16 = jnp.zeros((16,), jnp.int32)
        pltpu.sync_copy(info_hbm, infov)
        kc_lo = plsc.load_gather(infov, [lanes, zeros16])
        kc_hi = plsc.load_gather(infov, [lanes + 16, zeros16])
        pad_lo = ((kc_lo + 511) >> 9) << 9
        pad_hi = ((kc_hi + 511) >> 9) << 9
        off = (jnp.sum(jnp.where(lanes < g, pad_lo, 0))
               + jnp.sum(jnp.where(lanes + 16 < g, pad_hi, 0)))
        grow = plsc.load_gather(infov, [jnp.full((16,), g, jnp.int32), lanes])
        vk = jnp.sum(jnp.where(lanes == 1, grow, 0))
        ntie = jnp.sum(jnp.where(lanes == 2, grow, 0))
        dummyv = jnp.full((16,), DUMMY, jnp.int32)
        off = pl.multiple_of(off, 512)

        def chunk(i, carry):
            base = i * TOPK_CC
            pltpu.sync_copy(src_hbm.at[pl.ds(base, TOPK_CC)], srcv)
            pltpu.sync_copy(dst_hbm.at[pl.ds(base, TOPK_CC)], dstv)
            pltpu.sync_copy(pbits_hbm.at[pl.ds(base, TOPK_CC)], probv)

            def vreg(j, carry):
                cur, fl, tiecnt = carry
                s = srcv[pl.ds(j * 16, 16)]
                d = dstv[pl.ds(j * 16, 16)]
                pb = probv[pl.ds(j * 16, 16)]
                m = lax.shift_right_logical(s * GMUL, 16) == g
                gt = jnp.logical_and(m, pb > vk)
                tie = jnp.logical_and(m, pb == vk)
                tie_i = jnp.where(tie, 1, 0)
                pc = plsc.cumsum(tie_i)
                keep_tie = jnp.logical_and(tie, tiecnt + pc - 1 < ntie)
                keep = jnp.logical_or(gt, keep_tie)
                kcum = plsc.cumsum(jnp.where(keep, 1, 0))
                kidx = jnp.where(keep, cur + kcum - 1, jnp.int32(SCAP - 1))
                plsc.store_scatter(stag_s, [kidx], s)
                plsc.store_scatter(stag_d, [kidx], d)
                cur = cur + jnp.max(kcum)
                tiecnt = tiecnt + jnp.max(pc)

                def flush():
                    fo = pl.multiple_of(off + fl * 512, 512)
                    pltpu.sync_copy(stag_s.at[pl.ds(0, 512)],
                                    ksrc_out.at[pl.ds(fo, 512)])
                    pltpu.sync_copy(stag_d.at[pl.ds(0, 512)],
                                    kdst_out.at[pl.ds(fo, 512)])
                    stag_s[pl.ds(0, 16)] = stag_s[pl.ds(512, 16)]
                    stag_d[pl.ds(0, 16)] = stag_d[pl.ds(512, 16)]

                do = cur >= 512
                lax.cond(do, flush, lambda: None)
                cur = jnp.where(do, cur - 512, cur)
                fl = jnp.where(do, fl + 1, fl)
                return cur, fl, tiecnt

            return lax.fori_loop(0, TOPK_CC // 16, vreg, carry)

        cur, fl, _ = lax.fori_loop(0, NCH, chunk,
                                   (jnp.int32(0), jnp.int32(0), jnp.int32(0)))

        def padstore(j, cur):
            stag_s[pl.ds(cur + j * 16, 16)] = dummyv
            stag_d[pl.ds(cur + j * 16, 16)] = dummyv
            return cur

        lax.fori_loop(0, 32, padstore, cur)

        def final_flush():
            fo = pl.multiple_of(off + fl * 512, 512)
            pltpu.sync_copy(stag_s.at[pl.ds(0, 512)],
                            ksrc_out.at[pl.ds(fo, 512)])
            pltpu.sync_copy(stag_d.at[pl.ds(0, 512)],
                            kdst_out.at[pl.ds(fo, 512)])

        lax.cond(cur > 0, final_flush, lambda: None)

        total = (jnp.sum(pad_lo) + jnp.sum(pad_hi))
        tblk = total // 512

        def dfill(j, carry):
            stag_s[pl.ds(j * 16, 16)] = dummyv
            stag_d[pl.ds(j * 16, 16)] = dummyv
            return carry

        lax.fori_loop(0, 32, dfill, 0)

        def tail(j, carry):
            fo = pl.multiple_of((tblk + g + 32 * j) * 512, 512)
            pltpu.sync_copy(stag_s.at[pl.ds(0, 512)], ksrc_out.at[pl.ds(fo, 512)])
            pltpu.sync_copy(stag_d.at[pl.ds(0, 512)], kdst_out.at[pl.ds(fo, 512)])
            return carry

        nt = (KCAP // 512 - tblk - g + 31) // 32
        lax.fori_loop(0, nt, tail, 0)

    return emit


def _make_pair_agg_dyn(D):
    C = 128
    mesh = plsc.VectorSubcoreMesh(core_axis_name="c", subcore_axis_name="s")
    rows_per = NPAD // 16

    @functools.partial(
        pl.kernel, mesh=mesh,
        compiler_params=pltpu.CompilerParams(needs_layout_passes=False),
        out_type=[
            jax.ShapeDtypeStruct((2, NPAD, D), jnp.float32),
            jax.ShapeDtypeStruct((2, NPAD, 16), jnp.float32),
        ],
        scratch_types=[
            pltpu.VMEM((C,), jnp.int32),
            pltpu.VMEM((C,), jnp.int32),
            pltpu.VMEM((C, D), jnp.float32),
            pltpu.VMEM((C, D), jnp.float32),
            pltpu.VMEM((C, 16), jnp.float32),
            pltpu.VMEM((32, 16), jnp.int32),
            pltpu.VMEM_SHARED((NPAD, D), jnp.float32),
            pltpu.VMEM_SHARED((NPAD, 16), jnp.float32),
            pltpu.SemaphoreType.DMA,
            pltpu.SemaphoreType.DMA,
        ],
    )
    def agg(src_hbm, dst_hbm, x_hbm, info_hbm, zfeat_hbm, zcnt_hbm, ones_hbm,
            acc_out, cnt_out,
            idx_s, idx_d, rows_a, rows_b, ones_v, infov, acc_sh, cnt_sh, sem1, sem2):
        c = lax.axis_index("c")
        s = lax.axis_index("s")
        wid = c * 16 + s
        lanes = lax.iota(jnp.int32, 16)
        zeros16 = jnp.zeros((16,), jnp.int32)
        pltpu.sync_copy(info_hbm, infov)
        kc_lo = plsc.load_gather(infov, [lanes, zeros16])
        kc_hi = plsc.load_gather(infov, [lanes + 16, zeros16])
        total = (jnp.sum(((kc_lo + 511) >> 9) << 9)
                 + jnp.sum(((kc_hi + 511) >> 9) << 9))
        nchunks = total // C
        pltpu.sync_copy(zfeat_hbm.at[pl.ds(s * rows_per, rows_per)],
                        acc_sh.at[pl.ds(s * rows_per, rows_per)])
        pltpu.sync_copy(zcnt_hbm.at[pl.ds(s * rows_per, rows_per)],
                        cnt_sh.at[pl.ds(s * rows_per, rows_per)])
        pltpu.sync_copy(ones_hbm, ones_v)
        plsc.subcore_barrier()

        def chunk(i, carry):
            base = (wid + 32 * i) * C
            pltpu.sync_copy(src_hbm.at[pl.ds(base, C)], idx_s)
            pltpu.sync_copy(dst_hbm.at[pl.ds(base, C)], idx_d)
            cp1 = pltpu.async_copy(x_hbm.at[idx_s], rows_a, sem1)
            cp2 = pltpu.async_copy(x_hbm.at[idx_d], rows_b, sem2)
            cp1.wait()
            cp2.wait()
            pltpu.sync_copy(rows_a, acc_sh.at[idx_d], add=True)
            pltpu.sync_copy(rows_b, acc_sh.at[idx_s], add=True)
            pltpu.sync_copy(ones_v, cnt_sh.at[idx_d], add=True)
            pltpu.sync_copy(ones_v, cnt_sh.at[idx_s], add=True)
            return carry

        nmine = (nchunks - wid + 31) // 32
        lax.fori_loop(0, nmine, chunk, 0)
        plsc.subcore_barrier()
        pltpu.sync_copy(acc_sh.at[pl.ds(s * rows_per, rows_per)],
                        acc_out.at[c].at[pl.ds(s * rows_per, rows_per)])
        pltpu.sync_copy(cnt_sh.at[pl.ds(s * rows_per, rows_per)],
                        cnt_out.at[c].at[pl.ds(s * rows_per, rows_per)])

    return agg


def _make_edge_gather(D=128, C=320):
    mesh = plsc.VectorSubcoreMesh(core_axis_name="c", subcore_axis_name="s")
    PT = ET // 32
    NT = PT // C

    @functools.partial(
        pl.kernel, mesh=mesh,
        compiler_params=pltpu.CompilerParams(needs_layout_passes=False),
        out_type=[
            jax.ShapeDtypeStruct((ET, D), jnp.float32),
            jax.ShapeDtypeStruct((ET, D), jnp.float32),
        ],
        scratch_types=[
            pltpu.VMEM((C,), jnp.int32),
            pltpu.VMEM((C,), jnp.int32),
            pltpu.VMEM((C, D), jnp.float32),
            pltpu.VMEM((C, D), jnp.float32),
            pltpu.SemaphoreType.DMA,
            pltpu.SemaphoreType.DMA,
        ],
    )
    def gat(src_hbm, dst_hbm, z_hbm, s_out, d_out,
            idx_s, idx_d, rows_a, rows_b, sem1, sem2):
        wid = lax.axis_index("c") * 16 + lax.axis_index("s")
        base0 = wid * PT

        def chunk(i, carry):
            base = base0 + i * C
            pltpu.sync_copy(src_hbm.at[pl.ds(base, C)], idx_s)
            pltpu.sync_copy(dst_hbm.at[pl.ds(base, C)], idx_d)
            cp1 = pltpu.async_copy(z_hbm.at[idx_s], rows_a, sem1)
            cp2 = pltpu.async_copy(z_hbm.at[idx_d], rows_b, sem2)
            cp1.wait()
            cp2.wait()
            pltpu.sync_copy(rows_a, s_out.at[pl.ds(base, C)])
            pltpu.sync_copy(rows_b, d_out.at[pl.ds(base, C)])
            return carry

        lax.fori_loop(0, NT, chunk, 0)

    return gat


EDB = 2048


def _edge_decode_body(s_ref, d_ref, w1s_ref, w1a_ref, w1m_ref, b1_ref, g_ref,
                      beta_ref, w2_ref, b2_ref, logit_ref, prob_ref):
    i = pl.program_id(0)
    s = s_ref[:, :64]
    d = d_ref[:, :64]
    acc = (jnp.dot(s + d, w1s_ref[...], preferred_element_type=jnp.float32)
           + jnp.dot(jnp.abs(s - d), w1a_ref[...], preferred_element_type=jnp.float32)
           + jnp.dot(s * d, w1m_ref[...], preferred_element_type=jnp.float32)
           + b1_ref[...])
    mu = jnp.mean(acc, axis=-1, keepdims=True)
    var = jnp.mean((acc - mu) ** 2, axis=-1, keepdims=True)
    h = (acc - mu) / jnp.sqrt(var + 1e-5) * g_ref[...] + beta_ref[...]
    h = _mish(h)
    logit = jnp.sum(h * w2_ref[...], axis=-1, keepdims=True) + b2_ref[...]
    logit_ref[...] = logit
    prob_ref[...] = jax.nn.sigmoid(logit)
    del i


def _edge_decode_tc(s_rows, d_rows, p):
    W1 = p['ed_W1']
    w1s, w1a, w1m = W1[0:64], W1[64:128], W1[128:192]
    logit, prob = pl.pallas_call(
        _edge_decode_body,
        grid=(ET // EDB,),
        in_specs=[
            pl.BlockSpec((EDB, 128), lambda i: (i, 0)),
            pl.BlockSpec((EDB, 128), lambda i: (i, 0)),
            pl.BlockSpec((64, 64), lambda i: (0, 0)),
            pl.BlockSpec((64, 64), lambda i: (0, 0)),
            pl.BlockSpec((64, 64), lambda i: (0, 0)),
            pl.BlockSpec((64,), lambda i: (0,)),
            pl.BlockSpec((64,), lambda i: (0,)),
            pl.BlockSpec((64,), lambda i: (0,)),
            pl.BlockSpec((1, 64), lambda i: (0, 0)),
            pl.BlockSpec((1, 1), lambda i: (0, 0)),
        ],
        out_specs=[
            pl.BlockSpec((EDB, 1), lambda i: (i, 0)),
            pl.BlockSpec((EDB, 1), lambda i: (i, 0)),
        ],
        out_shape=[
            jax.ShapeDtypeStruct((ET, 1), jnp.float32),
            jax.ShapeDtypeStruct((ET, 1), jnp.float32),
        ],
    )(s_rows, d_rows, w1s, w1a, w1m, p['ed_b1'], p['ed_g'], p['ed_beta'],
      p['ed_W2'].reshape(1, 64), p['ed_b2'].reshape(1, 1))
    return logit[:, 0], prob


def _sage_body(acc_ref, cnt_ref, x_ref, wl_ref, bl_ref, wr_ref, o_ref):
    acc = acc_ref[0] + acc_ref[1]
    cnt = (cnt_ref[0] + cnt_ref[1])[:, 0:1]
    mean = acc / jnp.clip(cnt, 1.0)
    o_ref[...] = _mish(jnp.dot(mean, wl_ref[...], preferred_element_type=jnp.float32)
                       + bl_ref[...]
                       + jnp.dot(x_ref[...], wr_ref[...], preferred_element_type=jnp.float32))


def _sage_tc(acc2, cnt2, x_pad, Wl, bl, Wr):
    DO = Wl.shape[1]
    return pl.pallas_call(
        _sage_body,
        out_shape=jax.ShapeDtypeStruct((NPAD, DO), jnp.float32),
    )(acc2, cnt2, x_pad, Wl, bl, Wr)


def _heads_body(h_ref, nmw_ref, nmb_ref, nlw_ref, nlb_ref, epw_ref, epb_ref,
                emw_ref, emb_ref, elw_ref, elb_ref, n42_ref, n43_ref,
                zn_ref, ze_ref, kl_ref):
    h = h_ref[...]
    dot = lambda a, w: jnp.dot(a, w, preferred_element_type=jnp.float32)
    zpad = jnp.zeros((NPAD, 64), jnp.float32)
    n_mu = dot(h, nmw_ref[...]) + nmb_ref[...]
    n_lv = dot(h, nlw_ref[...]) + nlb_ref[...]
    zn_ref[...] = jnp.concatenate(
        [n_mu + n42_ref[...] * jnp.exp(0.5 * n_lv), zpad], axis=1)
    eh = _mish(dot(h, epw_ref[...]) + epb_ref[...])
    e_mu = dot(eh, emw_ref[...]) + emb_ref[...]
    e_lv = dot(eh, elw_ref[...]) + elb_ref[...]
    ze_ref[...] = jnp.concatenate(
        [e_mu + n43_ref[...] * jnp.exp(0.5 * e_lv), zpad], axis=1)
    rows = jax.lax.broadcasted_iota(jnp.int32, (NPAD, 1), 0)
    mask = rows < N_NODES
    s_e = jnp.sum(jnp.where(mask, 1.0 + e_lv - e_mu * e_mu - jnp.exp(e_lv), 0.0))
    s_n = jnp.sum(jnp.where(mask, 1.0 + n_lv - n_mu * n_mu - jnp.exp(n_lv), 0.0))
    col = jax.lax.broadcasted_iota(jnp.int32, (1, 2), 1)
    kl_ref[...] = jnp.where(col == 0, s_e, s_n)


def _heads_tc(h_pad, p, n42, n43):
    zn, ze, kl = pl.pallas_call(
        _heads_body,
        out_shape=[
            jax.ShapeDtypeStruct((NPAD, 128), jnp.float32),
            jax.ShapeDtypeStruct((NPAD, 128), jnp.float32),
            jax.ShapeDtypeStruct((1, 2), jnp.float32),
        ],
    )(h_pad, p['nm_W'], p['nm_b'], p['nl_W'], p['nl_b'], p['ep_W'], p['ep_b'],
      p['em_W'], p['em_b'], p['el_W'], p['el_b'], n42, n43)
    return zn, ze, kl


def _mish(x):
    return x * jnp.tanh(jax.nn.softplus(x))


def _layernorm(x, g, b):
    mu = jnp.mean(x, axis=-1, keepdims=True)
    var = jnp.mean((x - mu) ** 2, axis=-1, keepdims=True)
    return (x - mu) / jnp.sqrt(var + 1e-5) * g + b


def _sage(x, ei, Wl, bl, Wr):
    n = x.shape[0]
    src, dst = ei[0], ei[1]
    agg = jnp.zeros((n, x.shape[1]), x.dtype).at[dst].add(x[src])
    cnt = jnp.zeros((n,), x.dtype).at[dst].add(1.0)
    mean = agg / jnp.clip(cnt, 1.0)[:, None]
    return mean @ Wl + bl + x @ Wr


def _sage_w(x, ei, w, Wl, bl, Wr):
    n = x.shape[0]
    src, dst = ei[0], ei[1]
    agg = jnp.zeros((n, x.shape[1]), x.dtype).at[dst].add(x[src] * w[:, None])
    cnt = jnp.zeros((n,), x.dtype).at[dst].add(w)
    mean = agg / jnp.clip(cnt, 1.0)[:, None]
    return mean @ Wl + bl + x @ Wr


def _bce_logits(l, y):
    return jnp.mean(jnp.maximum(l, 0.0) - l * y + jnp.log1p(jnp.exp(-jnp.abs(l))))


def _edge_decode(z, ei, p):
    s = z[ei[0]]
    d = z[ei[1]]
    feat = jnp.concatenate([s + d, jnp.abs(s - d), s * d], axis=1)
    h = _mish(_layernorm(feat @ p['ed_W1'] + p['ed_b1'], p['ed_g'], p['ed_beta']))
    return (h @ p['ed_W2'] + p['ed_b2'])[:, 0]


def _mm_mish_body(a_ref, w_ref, b_ref, o_ref):
    acc = jnp.dot(a_ref[...], w_ref[...], preferred_element_type=jnp.float32)
    o_ref[...] = _mish(acc + b_ref[...])


def _mm_body(a_ref, w_ref, b_ref, o_ref):
    acc = jnp.dot(a_ref[...], w_ref[...], preferred_element_type=jnp.float32)
    o_ref[...] = acc + b_ref[...]


def _big_mlp(zf, W1, b1, W2, b2):
    Bn, K1 = zf.shape
    N1 = W1.shape[1]
    N2 = W2.shape[1]
    BN1 = 512
    BN2 = 512
    hid = pl.pallas_call(
        _mm_mish_body,
        grid=(pl.cdiv(N1, BN1),),
        in_specs=[
            pl.BlockSpec((Bn, K1), lambda j: (0, 0)),
            pl.BlockSpec((K1, BN1), lambda j: (0, j)),
            pl.BlockSpec((BN1,), lambda j: (j,)),
        ],
        out_specs=pl.BlockSpec((Bn, BN1), lambda j: (0, j)),
        out_shape=jax.ShapeDtypeStruct((Bn, N1), jnp.float32),
    )(zf, W1, b1)
    xr = pl.pallas_call(
        _mm_body,
        grid=(pl.cdiv(N2, BN2),),
        in_specs=[
            pl.BlockSpec((Bn, N1), lambda j: (0, 0)),
            pl.BlockSpec((N1, BN2), lambda j: (0, j)),
            pl.BlockSpec((BN2,), lambda j: (j,)),
        ],
        out_specs=pl.BlockSpec((Bn, BN2), lambda j: (0, j)),
        out_shape=jax.ShapeDtypeStruct((Bn, N2), jnp.float32),
    )(hid, W2, b2)
    return xr


def kernel(x, pos_edge_index, neg_edge_index, original_features, batch, y, sw_ratio, params):
    p = params
    Bn = y.shape[0]
    pos_src = pos_edge_index[0]
    pos_dst = pos_edge_index[1]
    x_pad = jnp.zeros((NPAD, 128), jnp.float32).at[:N_NODES, :x.shape[1]].set(x)
    acc2a, cnt2a = _pair_agg(pos_src, pos_dst, x_pad, 128)
    Wl1 = jnp.zeros((128, 128), jnp.float32).at[:116].set(p['c1_Wl'])
    Wr1 = jnp.zeros((128, 128), jnp.float32).at[:116].set(p['c1_Wr'])
    h1 = _sage_tc(acc2a, cnt2a, x_pad, Wl1, p['c1_bl'], Wr1)
    acc2b, cnt2b = _pair_agg(pos_src, pos_dst, h1, 128)
    h2 = _sage_tc(acc2b, cnt2b, h1, p['c2_Wl'], p['c2_bl'], p['c2_Wr'])
    n42 = jnp.zeros((NPAD, 64), jnp.float32).at[:N_NODES].set(
        jax.random.normal(jax.random.key(42), (N_NODES, 64), jnp.float32))
    n43 = jnp.zeros((NPAD, 64), jnp.float32).at[:N_NODES].set(
        jax.random.normal(jax.random.key(43), (N_NODES, 64), jnp.float32))
    znp, zep, klp = _heads_tc(h2, p, n42, n43)
    src_all = jnp.concatenate([pos_edge_index[0], neg_edge_index[0]])
    dst_all = jnp.concatenate([pos_edge_index[1], neg_edge_index[1]])
    s_rows, d_rows = _make_edge_gather()(src_all, dst_all, zep)
    logits, probs2d = _edge_decode_tc(s_rows, d_rows, p)
    labels = jnp.concatenate([jnp.ones((ET // 2,), jnp.float32),
                              jnp.zeros((ET // 2,), jnp.float32)])
    loss_edges = _bce_logits(logits, labels)
    pbits = lax.bitcast_convert_type(probs2d[:, 0], jnp.int32)
    info = _make_topk_stats()(src_all, pbits)
    if isinstance(info, (tuple, list)):
        info = info[0]
    ksrc, kdst = _make_topk_emit()(src_all, dst_all, pbits, info)
    pad_wb = lambda W, b: (
        jnp.zeros((128, 128), jnp.float32).at[:64, :64].set(W),
        jnp.zeros((128,), jnp.float32).at[:64].set(b))
    d1_Wl, d1_bl = pad_wb(p['d1_Wl'], p['d1_bl'])
    d1_Wr, _ = pad_wb(p['d1_Wr'], p['d1_bl'])
    d2_Wl, d2_bl = pad_wb(p['d2_Wl'], p['d2_bl'])
    d2_Wr, _ = pad_wb(p['d2_Wr'], p['d2_bl'])
    accC, cntC = _pair_agg(ksrc, kdst, znp, 128)
    z1 = _sage_tc(accC, cntC, znp, d1_Wl, d1_bl, d1_Wr)
    accD, cntD = _pair_agg(ksrc, kdst, z1, 128)
    z2 = _sage_tc(accD, cntD, z1, d2_Wl, d2_bl, d2_Wr)
    zf = z2[:N_NODES, :64].reshape(Bn, NUM_NODES * 64)
    xr = _big_mlp(zf, p['f1_W'], p['f1_b'], p['f2_W'], p['f2_b']).reshape(Bn, NUM_NODES, -1)
    sym = (xr + jnp.transpose(xr, (0, 2, 1))) / 2.0
    eye = jnp.eye(NUM_NODES, dtype=bool)[None, :, :]
    x_rec = jnp.where(eye, 1.0, sym).reshape(Bn * NUM_NODES, -1)
    loss_nodes = jnp.mean((x_rec - original_features) ** 2)
    denom = jnp.float32(N_NODES * 64)
    kl = 0.5 * (-0.5 * klp[0, 0] / denom + -0.5 * klp[0, 1] / denom)
    total = loss_edges + loss_nodes + 0.1 * kl
    return (total, logits, labels, x_rec)


# speedup vs baseline: 2.4295x; 2.4295x over previous
import functools

import jax
import jax.numpy as jnp
import numpy as np
from jax import lax
from jax.experimental import pallas as pl
from jax.experimental.pallas import tpu as pltpu
from jax.experimental.pallas import tpu_sc as plsc

NUM_NODES = 116
KEEP_RATIO = 0.6
N_NODES = 3712
NPAD = 3840
DUMMY = 3712


def _make_pair_agg(P, D, C):
    assert P % (32 * C) == 0
    NT = P // (32 * C)
    mesh = plsc.VectorSubcoreMesh(core_axis_name="c", subcore_axis_name="s")
    rows_per = NPAD // 16

    @functools.partial(
        pl.kernel, mesh=mesh,
        compiler_params=pltpu.CompilerParams(needs_layout_passes=False),
        out_type=[
            jax.ShapeDtypeStruct((2, NPAD, D), jnp.float32),
            jax.ShapeDtypeStruct((2, NPAD, 16), jnp.float32),
        ],
        scratch_types=[
            pltpu.VMEM((C,), jnp.int32),
            pltpu.VMEM((C,), jnp.int32),
            pltpu.VMEM((C, D), jnp.float32),
            pltpu.VMEM((C, D), jnp.float32),
            pltpu.VMEM((C, 16), jnp.float32),
            pltpu.VMEM_SHARED((NPAD, D), jnp.float32),
            pltpu.VMEM_SHARED((NPAD, 16), jnp.float32),
            pltpu.SemaphoreType.DMA,
            pltpu.SemaphoreType.DMA,
        ],
    )
    def agg(src_hbm, dst_hbm, x_hbm, zfeat_hbm, zcnt_hbm, ones_hbm,
            acc_out, cnt_out,
            idx_s, idx_d, rows_a, rows_b, ones_v, acc_sh, cnt_sh, sem1, sem2):
        c = lax.axis_index("c")
        s = lax.axis_index("s")
        wid = c * 16 + s
        pltpu.sync_copy(zfeat_hbm.at[pl.ds(s * rows_per, rows_per)],
                        acc_sh.at[pl.ds(s * rows_per, rows_per)])
        pltpu.sync_copy(zcnt_hbm.at[pl.ds(s * rows_per, rows_per)],
                        cnt_sh.at[pl.ds(s * rows_per, rows_per)])
        pltpu.sync_copy(ones_hbm, ones_v)
        plsc.subcore_barrier()
        base0 = wid * (NT * C)

        def chunk(i, carry):
            base = base0 + i * C
            pltpu.sync_copy(src_hbm.at[pl.ds(base, C)], idx_s)
            pltpu.sync_copy(dst_hbm.at[pl.ds(base, C)], idx_d)
            cp1 = pltpu.async_copy(x_hbm.at[idx_s], rows_a, sem1)
            cp2 = pltpu.async_copy(x_hbm.at[idx_d], rows_b, sem2)
            cp1.wait()
            cp2.wait()
            pltpu.sync_copy(rows_a, acc_sh.at[idx_d], add=True)
            pltpu.sync_copy(rows_b, acc_sh.at[idx_s], add=True)
            pltpu.sync_copy(ones_v, cnt_sh.at[idx_d], add=True)
            pltpu.sync_copy(ones_v, cnt_sh.at[idx_s], add=True)
            return carry

        lax.fori_loop(0, NT, chunk, 0)
        plsc.subcore_barrier()
        pltpu.sync_copy(acc_sh.at[pl.ds(s * rows_per, rows_per)],
                        acc_out.at[c].at[pl.ds(s * rows_per, rows_per)])
        pltpu.sync_copy(cnt_sh.at[pl.ds(s * rows_per, rows_per)],
                        cnt_out.at[c].at[pl.ds(s * rows_per, rows_per)])

    return agg


def _pair_agg(src, dst, x_pad, D, C=160):
    P = src.shape[0]
    zfeat = jnp.zeros((NPAD, D), jnp.float32)
    zcnt = jnp.zeros((NPAD, 16), jnp.float32)
    ones = jnp.zeros((C, 16), jnp.float32).at[:, 0].set(1.0)
    return _make_pair_agg(P, D, C)(src, dst, x_pad, zfeat, zcnt, ones)


ET = 102400
GMUL = 565
TOPK_CC = 2048
KCAP = 81920
SENT = -2147483648


def _count_ge(pbuf, nv, t):
    def body(i, acc):
        v = pbuf[pl.ds(i * 16, 16)]
        return acc + jnp.where(v >= t, 1, 0)

    accv = lax.fori_loop(0, nv, body, jnp.zeros((16,), jnp.int32))
    return jnp.sum(accv)


def _make_topk_stats():
    mesh = plsc.VectorSubcoreMesh(core_axis_name="c", subcore_axis_name="s")
    NCH = ET // TOPK_CC

    @functools.partial(
        pl.kernel, mesh=mesh,
        compiler_params=pltpu.CompilerParams(needs_layout_passes=False),
        out_type=[jax.ShapeDtypeStruct((32, 16), jnp.int32)],
        scratch_types=[
            pltpu.VMEM((ET + 32,), jnp.int32),
            pltpu.VMEM((TOPK_CC,), jnp.int32),
            pltpu.VMEM((TOPK_CC,), jnp.int32),
            pltpu.VMEM((16,), jnp.int32),
        ],
    )
    def stats(src_hbm, pbits_hbm, info_out, pbuf, srcv, probv, infov):
        g = lax.axis_index("c") * 16 + lax.axis_index("s")
        lanes = lax.iota(jnp.int32, 16)
        dump = jnp.int32(ET + 31)

        def chunk(i, cnt):
            base = i * TOPK_CC
            pltpu.sync_copy(src_hbm.at[pl.ds(base, TOPK_CC)], srcv)
            pltpu.sync_copy(pbits_hbm.at[pl.ds(base, TOPK_CC)], probv)

            def vreg(j, cnt):
                s = srcv[pl.ds(j * 16, 16)]
                pb = probv[pl.ds(j * 16, 16)]
                m = lax.shift_right_logical(s * GMUL, 16) == g
                pc = plsc.cumsum(jnp.where(m, 1, 0))
                plsc.store_scatter(pbuf, [jnp.where(m, cnt + pc - 1, dump)], pb)
                return cnt + jnp.max(pc)

            return lax.fori_loop(0, TOPK_CC // 16, vreg, cnt)

        cnt = lax.fori_loop(0, NCH, chunk, jnp.int32(0))
        pbuf[pl.ds(cnt, 16)] = jnp.full((16,), SENT, jnp.int32)
        nv = (cnt + 15) // 16
        k = jnp.maximum(1, (jnp.float32(KEEP_RATIO) * cnt.astype(jnp.float32)).astype(jnp.int32))

        def bstep(_, lohi):
            lo, hi = lohi
            mid = (lo + hi) // 2
            go = hi - lo > 1
            c = _count_ge(pbuf, nv, mid)
            take = jnp.logical_and(go, c >= k)
            lo2 = jnp.where(take, mid, lo)
            hi2 = jnp.where(jnp.logical_and(go, jnp.logical_not(c >= k)), mid, hi)
            return lo2, hi2

        vk, _ = lax.fori_loop(0, 31, bstep, (jnp.int32(0), jnp.int32(0x3F800001)))
        c_gt = _count_ge(pbuf, nv, vk + 1)
        kc = jnp.where(cnt > 0, k, 0)
        ntie = k - c_gt
        row = jnp.where(lanes == 0, kc,
                        jnp.where(lanes == 1, vk,
                                  jnp.where(lanes == 2, ntie, 0)))
        infov[...] = row
        pltpu.sync_copy(infov, info_out.at[g])

    return stats


def _make_topk_emit():
    mesh = plsc.VectorSubcoreMesh(core_axis_name="c", subcore_axis_name="s")
    NCH = ET // TOPK_CC
    SCAP = 1040

    @functools.partial(
        pl.kernel, mesh=mesh,
        compiler_params=pltpu.CompilerParams(needs_layout_passes=False),
        out_type=[
            jax.ShapeDtypeStruct((KCAP,), jnp.int32),
            jax.ShapeDtypeStruct((KCAP,), jnp.int32),
        ],
        scratch_types=[
            pltpu.VMEM((TOPK_CC,), jnp.int32),
            pltpu.VMEM((TOPK_CC,), jnp.int32),
            pltpu.VMEM((TOPK_CC,), jnp.int32),
            pltpu.VMEM((SCAP,), jnp.int32),
            pltpu.VMEM((SCAP,), jnp.int32),
            pltpu.VMEM((32, 16), jnp.int32),
        ],
    )
    def emit(src_hbm, dst_hbm, pbits_hbm, info_hbm, ksrc_out, kdst_out,
             srcv, dstv, probv, stag_s, stag_d, infov):
        g = lax.axis_index("c") * 16 + lax.axis_index("s")
        lanes = lax.iota(jnp.int32, 16)
        zeros16 = jnp.zeros((16,), jnp.int32)
        pltpu.sync_copy(info_hbm, infov)
        kc_lo = plsc.load_gather(infov, [lanes, zeros16])
        kc_hi = plsc.load_gather(infov, [lanes + 16, zeros16])
        pad_lo = ((kc_lo + 511) >> 9) << 9
        pad_hi = ((kc_hi + 511) >> 9) << 9
        off = (jnp.sum(jnp.where(lanes < g, pad_lo, 0))
               + jnp.sum(jnp.where(lanes + 16 < g, pad_hi, 0)))
        grow = plsc.load_gather(infov, [jnp.full((16,), g, jnp.int32), lanes])
        vk = jnp.sum(jnp.where(lanes == 1, grow, 0))
        ntie = jnp.sum(jnp.where(lanes == 2, grow, 0))
        dummyv = jnp.full((16,), DUMMY, jnp.int32)
        off = pl.multiple_of(off, 512)

        def chunk(i, carry):
            base = i * TOPK_CC
            pltpu.sync_copy(src_hbm.at[pl.ds(base, TOPK_CC)], srcv)
            pltpu.sync_copy(dst_hbm.at[pl.ds(base, TOPK_CC)], dstv)
            pltpu.sync_copy(pbits_hbm.at[pl.ds(base, TOPK_CC)], probv)

            def vreg(j, carry):
                cur, fl, tiecnt = carry
                s = srcv[pl.ds(j * 16, 16)]
                d = dstv[pl.ds(j * 16, 16)]
                pb = probv[pl.ds(j * 16, 16)]
                m = lax.shift_right_logical(s * GMUL, 16) == g
                gt = jnp.logical_and(m, pb > vk)
                tie = jnp.logical_and(m, pb == vk)
                tie_i = jnp.where(tie, 1, 0)
                pc = plsc.cumsum(tie_i)
                keep_tie = jnp.logical_and(tie, tiecnt + pc - 1 < ntie)
                keep = jnp.logical_or(gt, keep_tie)
                kcum = plsc.cumsum(jnp.where(keep, 1, 0))
                kidx = jnp.where(keep, cur + kcum - 1, jnp.int32(SCAP - 1))
                plsc.store_scatter(stag_s, [kidx], s)
                plsc.store_scatter(stag_d, [kidx], d)
                cur = cur + jnp.max(kcum)
                tiecnt = tiecnt + jnp.max(pc)

                def flush():
                    fo = pl.multiple_of(off + fl * 512, 512)
                    pltpu.sync_copy(stag_s.at[pl.ds(0, 512)],
                                    ksrc_out.at[pl.ds(fo, 512)])
                    pltpu.sync_copy(stag_d.at[pl.ds(0, 512)],
                                    kdst_out.at[pl.ds(fo, 512)])
                    stag_s[pl.ds(0, 16)] = stag_s[pl.ds(512, 16)]
                    stag_d[pl.ds(0, 16)] = stag_d[pl.ds(512, 16)]

                do = cur >= 512
                lax.cond(do, flush, lambda: None)
                cur = jnp.where(do, cur - 512, cur)
                fl = jnp.where(do, fl + 1, fl)
                return cur, fl, tiecnt

            return lax.fori_loop(0, TOPK_CC // 16, vreg, carry)

        cur, fl, _ = lax.fori_loop(0, NCH, chunk,
                                   (jnp.int32(0), jnp.int32(0), jnp.int32(0)))

        def padstore(j, cur):
            stag_s[pl.ds(cur + j * 16, 16)] = dummyv
            stag_d[pl.ds(cur + j * 16, 16)] = dummyv
            return cur

        lax.fori_loop(0, 32, padstore, cur)

        def final_flush():
            fo = pl.multiple_of(off + fl * 512, 512)
            pltpu.sync_copy(stag_s.at[pl.ds(0, 512)],
                            ksrc_out.at[pl.ds(fo, 512)])
            pltpu.sync_copy(stag_d.at[pl.ds(0, 512)],
                            kdst_out.at[pl.ds(fo, 512)])

        lax.cond(cur > 0, final_flush, lambda: None)

    return emit


def _make_pair_agg_dyn(D):
    C = 512
    mesh = plsc.VectorSubcoreMesh(core_axis_name="c", subcore_axis_name="s")
    rows_per = NPAD // 16

    @functools.partial(
        pl.kernel, mesh=mesh,
        compiler_params=pltpu.CompilerParams(needs_layout_passes=False,
                                             use_tc_tiling_on_sc=False),
        out_type=[
            jax.ShapeDtypeStruct((2, NPAD, D), jnp.float32),
            jax.ShapeDtypeStruct((2, NPAD, 16), jnp.float32),
        ],
        scratch_types=[
            pltpu.VMEM((C,), jnp.int32),
            pltpu.VMEM((C,), jnp.int32),
            pltpu.VMEM((C, D), jnp.float32),
            pltpu.VMEM((C, D), jnp.float32),
            pltpu.VMEM((C, 16), jnp.float32),
            pltpu.VMEM((32, 16), jnp.int32),
            pltpu.VMEM_SHARED((NPAD, D), jnp.float32),
            pltpu.VMEM_SHARED((NPAD, 16), jnp.float32),
            pltpu.SemaphoreType.DMA,
            pltpu.SemaphoreType.DMA,
        ],
    )
    def agg(src_hbm, dst_hbm, x_hbm, info_hbm, zfeat_hbm, zcnt_hbm, ones_hbm,
            acc_out, cnt_out,
            idx_s, idx_d, rows_a, rows_b, ones_v, infov, acc_sh, cnt_sh, sem1, sem2):
        c = lax.axis_index("c")
        s = lax.axis_index("s")
        wid = c * 16 + s
        lanes = lax.iota(jnp.int32, 16)
        zeros16 = jnp.zeros((16,), jnp.int32)
        pltpu.sync_copy(info_hbm, infov)
        kc_lo = plsc.load_gather(infov, [lanes, zeros16])
        kc_hi = plsc.load_gather(infov, [lanes + 16, zeros16])
        total = (jnp.sum(((kc_lo + 511) >> 9) << 9)
                 + jnp.sum(((kc_hi + 511) >> 9) << 9))
        nchunks = total // C
        pltpu.sync_copy(zfeat_hbm.at[pl.ds(s * rows_per, rows_per)],
                        acc_sh.at[pl.ds(s * rows_per, rows_per)])
        pltpu.sync_copy(zcnt_hbm.at[pl.ds(s * rows_per, rows_per)],
                        cnt_sh.at[pl.ds(s * rows_per, rows_per)])
        pltpu.sync_copy(ones_hbm, ones_v)
        plsc.subcore_barrier()

        def chunk(i, carry):
            base = (wid + 32 * i) * C
            pltpu.sync_copy(src_hbm.at[pl.ds(base, C)], idx_s)
            pltpu.sync_copy(dst_hbm.at[pl.ds(base, C)], idx_d)
            cp1 = pltpu.async_copy(x_hbm.at[idx_s], rows_a, sem1)
            cp2 = pltpu.async_copy(x_hbm.at[idx_d], rows_b, sem2)
            cp1.wait()
            cp2.wait()
            pltpu.sync_copy(rows_a, acc_sh.at[idx_d], add=True)
            pltpu.sync_copy(rows_b, acc_sh.at[idx_s], add=True)
            pltpu.sync_copy(ones_v, cnt_sh.at[idx_d], add=True)
            pltpu.sync_copy(ones_v, cnt_sh.at[idx_s], add=True)
            return carry

        nmine = (nchunks - wid + 31) // 32
        lax.fori_loop(0, nmine, chunk, 0)
        plsc.subcore_barrier()
        pltpu.sync_copy(acc_sh.at[pl.ds(s * rows_per, rows_per)],
                        acc_out.at[c].at[pl.ds(s * rows_per, rows_per)])
        pltpu.sync_copy(cnt_sh.at[pl.ds(s * rows_per, rows_per)],
                        cnt_out.at[c].at[pl.ds(s * rows_per, rows_per)])

    return agg


def _make_edge_gather(D=128, C=200):
    mesh = plsc.VectorSubcoreMesh(core_axis_name="c", subcore_axis_name="s")
    PT = ET // 32
    NT = PT // C

    @functools.partial(
        pl.kernel, mesh=mesh,
        compiler_params=pltpu.CompilerParams(needs_layout_passes=False),
        out_type=[
            jax.ShapeDtypeStruct((ET, D), jnp.float32),
            jax.ShapeDtypeStruct((ET, D), jnp.float32),
        ],
        scratch_types=[
            pltpu.VMEM((C,), jnp.int32),
            pltpu.VMEM((C,), jnp.int32),
            pltpu.VMEM((C, D), jnp.float32),
            pltpu.VMEM((C, D), jnp.float32),
            pltpu.SemaphoreType.DMA,
            pltpu.SemaphoreType.DMA,
        ],
    )
    def gat(src_hbm, dst_hbm, z_hbm, s_out, d_out,
            idx_s, idx_d, rows_a, rows_b, sem1, sem2):
        wid = lax.axis_index("c") * 16 + lax.axis_index("s")
        base0 = wid * PT

        def chunk(i, carry):
            base = base0 + i * C
            pltpu.sync_copy(src_hbm.at[pl.ds(base, C)], idx_s)
            pltpu.sync_copy(dst_hbm.at[pl.ds(base, C)], idx_d)
            cp1 = pltpu.async_copy(z_hbm.at[idx_s], rows_a, sem1)
            cp2 = pltpu.async_copy(z_hbm.at[idx_d], rows_b, sem2)
            cp1.wait()
            cp2.wait()
            pltpu.sync_copy(rows_a, s_out.at[pl.ds(base, C)])
            pltpu.sync_copy(rows_b, d_out.at[pl.ds(base, C)])
            return carry

        lax.fori_loop(0, NT, chunk, 0)

    return gat


EDB = 2048


def _edge_decode_body(s_ref, d_ref, w1s_ref, w1a_ref, w1m_ref, b1_ref, g_ref,
                      beta_ref, w2_ref, b2_ref, logit_ref, prob_ref):
    i = pl.program_id(0)
    s = s_ref[:, :64]
    d = d_ref[:, :64]
    acc = (jnp.dot(s + d, w1s_ref[...], preferred_element_type=jnp.float32)
           + jnp.dot(jnp.abs(s - d), w1a_ref[...], preferred_element_type=jnp.float32)
           + jnp.dot(s * d, w1m_ref[...], preferred_element_type=jnp.float32)
           + b1_ref[...])
    mu = jnp.mean(acc, axis=-1, keepdims=True)
    var = jnp.mean((acc - mu) ** 2, axis=-1, keepdims=True)
    h = (acc - mu) / jnp.sqrt(var + 1e-5) * g_ref[...] + beta_ref[...]
    h = _mish(h)
    logit = jnp.sum(h * w2_ref[...], axis=-1, keepdims=True) + b2_ref[...]
    logit_ref[...] = logit
    prob_ref[...] = jax.nn.sigmoid(logit)
    del i


def _edge_decode_tc(s_rows, d_rows, p):
    W1 = p['ed_W1']
    w1s, w1a, w1m = W1[0:64], W1[64:128], W1[128:192]
    logit, prob = pl.pallas_call(
        _edge_decode_body,
        grid=(ET // EDB,),
        in_specs=[
            pl.BlockSpec((EDB, 128), lambda i: (i, 0)),
            pl.BlockSpec((EDB, 128), lambda i: (i, 0)),
            pl.BlockSpec((64, 64), lambda i: (0, 0)),
            pl.BlockSpec((64, 64), lambda i: (0, 0)),
            pl.BlockSpec((64, 64), lambda i: (0, 0)),
            pl.BlockSpec((64,), lambda i: (0,)),
            pl.BlockSpec((64,), lambda i: (0,)),
            pl.BlockSpec((64,), lambda i: (0,)),
            pl.BlockSpec((1, 64), lambda i: (0, 0)),
            pl.BlockSpec((1, 1), lambda i: (0, 0)),
        ],
        out_specs=[
            pl.BlockSpec((EDB, 1), lambda i: (i, 0)),
            pl.BlockSpec((EDB, 1), lambda i: (i, 0)),
        ],
        out_shape=[
            jax.ShapeDtypeStruct((ET, 1), jnp.float32),
            jax.ShapeDtypeStruct((ET, 1), jnp.float32),
        ],
    )(s_rows, d_rows, w1s, w1a, w1m, p['ed_b1'], p['ed_g'], p['ed_beta'],
      p['ed_W2'].reshape(1, 64), p['ed_b2'].reshape(1, 1))
    return logit[:, 0], prob


def _sage_body(acc_ref, cnt_ref, x_ref, wl_ref, bl_ref, wr_ref, o_ref):
    acc = acc_ref[0] + acc_ref[1]
    cnt = (cnt_ref[0] + cnt_ref[1])[:, 0:1]
    mean = acc / jnp.clip(cnt, 1.0)
    o_ref[...] = _mish(jnp.dot(mean, wl_ref[...], preferred_element_type=jnp.float32)
                       + bl_ref[...]
                       + jnp.dot(x_ref[...], wr_ref[...], preferred_element_type=jnp.float32))


def _sage_tc(acc2, cnt2, x_pad, Wl, bl, Wr):
    DO = Wl.shape[1]
    return pl.pallas_call(
        _sage_body,
        out_shape=jax.ShapeDtypeStruct((NPAD, DO), jnp.float32),
    )(acc2, cnt2, x_pad, Wl, bl, Wr)


def _heads_body(h_ref, nmw_ref, nmb_ref, nlw_ref, nlb_ref, epw_ref, epb_ref,
                emw_ref, emb_ref, elw_ref, elb_ref, n42_ref, n43_ref,
                zn_ref, ze_ref, kl_ref):
    h = h_ref[...]
    dot = lambda a, w: jnp.dot(a, w, preferred_element_type=jnp.float32)
    n_mu = dot(h, nmw_ref[...]) + nmb_ref[...]
    n_lv = dot(h, nlw_ref[...]) + nlb_ref[...]
    zn_ref[...] = n_mu + n42_ref[...] * jnp.exp(0.5 * n_lv)
    eh = _mish(dot(h, epw_ref[...]) + epb_ref[...])
    e_mu = dot(eh, emw_ref[...]) + emb_ref[...]
    e_lv = dot(eh, elw_ref[...]) + elb_ref[...]
    ze_ref[...] = e_mu + n43_ref[...] * jnp.exp(0.5 * e_lv)
    rows = jax.lax.broadcasted_iota(jnp.int32, (NPAD, 1), 0)
    mask = rows < N_NODES
    s_e = jnp.sum(jnp.where(mask, 1.0 + e_lv - e_mu * e_mu - jnp.exp(e_lv), 0.0))
    s_n = jnp.sum(jnp.where(mask, 1.0 + n_lv - n_mu * n_mu - jnp.exp(n_lv), 0.0))
    col = jax.lax.broadcasted_iota(jnp.int32, (1, 2), 1)
    kl_ref[...] = jnp.where(col == 0, s_e, s_n)


def _heads_tc(h_pad, p, n42, n43):
    zn, ze, kl = pl.pallas_call(
        _heads_body,
        out_shape=[
            jax.ShapeDtypeStruct((NPAD, 64), jnp.float32),
            jax.ShapeDtypeStruct((NPAD, 64), jnp.float32),
            jax.ShapeDtypeStruct((1, 2), jnp.float32),
        ],
    )(h_pad, p['nm_W'], p['nm_b'], p['nl_W'], p['nl_b'], p['ep_W'], p['ep_b'],
      p['em_W'], p['em_b'], p['el_W'], p['el_b'], n42, n43)
    return zn, ze, kl


def _mish(x):
    return x * jnp.tanh(jax.nn.softplus(x))


def _layernorm(x, g, b):
    mu = jnp.mean(x, axis=-1, keepdims=True)
    var = jnp.mean((x - mu) ** 2, axis=-1, keepdims=True)
    return (x - mu) / jnp.sqrt(var + 1e-5) * g + b


def _sage(x, ei, Wl, bl, Wr):
    n = x.shape[0]
    src, dst = ei[0], ei[1]
    agg = jnp.zeros((n, x.shape[1]), x.dtype).at[dst].add(x[src])
    cnt = jnp.zeros((n,), x.dtype).at[dst].add(1.0)
    mean = agg / jnp.clip(cnt, 1.0)[:, None]
    return mean @ Wl + bl + x @ Wr


def _sage_w(x, ei, w, Wl, bl, Wr):
    n = x.shape[0]
    src, dst = ei[0], ei[1]
    agg = jnp.zeros((n, x.shape[1]), x.dtype).at[dst].add(x[src] * w[:, None])
    cnt = jnp.zeros((n,), x.dtype).at[dst].add(w)
    mean = agg / jnp.clip(cnt, 1.0)[:, None]
    return mean @ Wl + bl + x @ Wr


def _bce_logits(l, y):
    return jnp.mean(jnp.maximum(l, 0.0) - l * y + jnp.log1p(jnp.exp(-jnp.abs(l))))


def _edge_decode(z, ei, p):
    s = z[ei[0]]
    d = z[ei[1]]
    feat = jnp.concatenate([s + d, jnp.abs(s - d), s * d], axis=1)
    h = _mish(_layernorm(feat @ p['ed_W1'] + p['ed_b1'], p['ed_g'], p['ed_beta']))
    return (h @ p['ed_W2'] + p['ed_b2'])[:, 0]


def _mm_mish_body(a_ref, w_ref, b_ref, o_ref):
    acc = jnp.dot(a_ref[...], w_ref[...], preferred_element_type=jnp.float32)
    o_ref[...] = _mish(acc + b_ref[...])


def _mm_body(a_ref, w_ref, b_ref, o_ref):
    acc = jnp.dot(a_ref[...], w_ref[...], preferred_element_type=jnp.float32)
    o_ref[...] = acc + b_ref[...]


def _big_mlp(zf, W1, b1, W2, b2):
    Bn, K1 = zf.shape
    N1 = W1.shape[1]
    N2 = W2.shape[1]
    BN1 = 512
    BN2 = 512
    hid = pl.pallas_call(
        _mm_mish_body,
        grid=(pl.cdiv(N1, BN1),),
        in_specs=[
            pl.BlockSpec((Bn, K1), lambda j: (0, 0)),
            pl.BlockSpec((K1, BN1), lambda j: (0, j)),
            pl.BlockSpec((BN1,), lambda j: (j,)),
        ],
        out_specs=pl.BlockSpec((Bn, BN1), lambda j: (0, j)),
        out_shape=jax.ShapeDtypeStruct((Bn, N1), jnp.float32),
    )(zf, W1, b1)
    xr = pl.pallas_call(
        _mm_body,
        grid=(pl.cdiv(N2, BN2),),
        in_specs=[
            pl.BlockSpec((Bn, N1), lambda j: (0, 0)),
            pl.BlockSpec((N1, BN2), lambda j: (0, j)),
            pl.BlockSpec((BN2,), lambda j: (j,)),
        ],
        out_specs=pl.BlockSpec((Bn, BN2), lambda j: (0, j)),
        out_shape=jax.ShapeDtypeStruct((Bn, N2), jnp.float32),
    )(hid, W2, b2)
    return xr


def kernel(x, pos_edge_index, neg_edge_index, original_features, batch, y, sw_ratio, params):
    p = params
    Bn = y.shape[0]
    pos_src = pos_edge_index[0]
    pos_dst = pos_edge_index[1]
    x_pad = jnp.zeros((NPAD, 128), jnp.float32).at[:N_NODES, :x.shape[1]].set(x)
    acc2a, cnt2a = _pair_agg(pos_src, pos_dst, x_pad, 128)
    Wl1 = jnp.zeros((128, 128), jnp.float32).at[:116].set(p['c1_Wl'])
    Wr1 = jnp.zeros((128, 128), jnp.float32).at[:116].set(p['c1_Wr'])
    h1 = _sage_tc(acc2a, cnt2a, x_pad, Wl1, p['c1_bl'], Wr1)
    acc2b, cnt2b = _pair_agg(pos_src, pos_dst, h1, 128)
    h2 = _sage_tc(acc2b, cnt2b, h1, p['c2_Wl'], p['c2_bl'], p['c2_Wr'])
    n42 = jnp.zeros((NPAD, 64), jnp.float32).at[:N_NODES].set(
        jax.random.normal(jax.random.key(42), (N_NODES, 64), jnp.float32))
    n43 = jnp.zeros((NPAD, 64), jnp.float32).at[:N_NODES].set(
        jax.random.normal(jax.random.key(43), (N_NODES, 64), jnp.float32))
    znp, zep, klp = _heads_tc(h2, p, n42, n43)
    src_all = jnp.concatenate([pos_edge_index[0], neg_edge_index[0]])
    dst_all = jnp.concatenate([pos_edge_index[1], neg_edge_index[1]])
    zep128 = jnp.zeros((NPAD, 128), jnp.float32).at[:, :64].set(zep)
    s_rows, d_rows = _make_edge_gather()(src_all, dst_all, zep128)
    logits, probs2d = _edge_decode_tc(s_rows, d_rows, p)
    labels = jnp.concatenate([jnp.ones((ET // 2,), jnp.float32),
                              jnp.zeros((ET // 2,), jnp.float32)])
    loss_edges = _bce_logits(logits, labels)
    pbits = lax.bitcast_convert_type(probs2d[:, 0], jnp.int32)
    info = _make_topk_stats()(src_all, pbits)
    if isinstance(info, (tuple, list)):
        info = info[0]
    ksrc, kdst = _make_topk_emit()(src_all, dst_all, pbits, info)
    z64 = jnp.zeros((NPAD, 64), jnp.float32)
    z16 = jnp.zeros((NPAD, 16), jnp.float32)
    ones512 = jnp.zeros((512, 16), jnp.float32).at[:, 0].set(1.0)
    accC, cntC = _make_pair_agg_dyn(64)(ksrc, kdst, znp, info, z64, z16, ones512)
    z1 = _sage_tc(accC, cntC, znp, p['d1_Wl'], p['d1_bl'], p['d1_Wr'])
    accD, cntD = _make_pair_agg_dyn(64)(ksrc, kdst, z1, info, z64, z16, ones512)
    z2 = _sage_tc(accD, cntD, z1, p['d2_Wl'], p['d2_bl'], p['d2_Wr'])
    zf = z2[:N_NODES].reshape(Bn, NUM_NODES * 64)
    xr = _big_mlp(zf, p['f1_W'], p['f1_b'], p['f2_W'], p['f2_b']).reshape(Bn, NUM_NODES, -1)
    sym = (xr + jnp.transpose(xr, (0, 2, 1))) / 2.0
    eye = jnp.eye(NUM_NODES, dtype=bool)[None, :, :]
    x_rec = jnp.where(eye, 1.0, sym).reshape(Bn * NUM_NODES, -1)
    loss_nodes = jnp.mean((x_rec - original_features) ** 2)
    denom = jnp.float32(N_NODES * 64)
    kl = 0.5 * (-0.5 * klp[0, 0] / denom + -0.5 * klp[0, 1] / denom)
    total = loss_edges + loss_nodes + 0.1 * kl
    return (total, logits, labels, x_rec)
